# Initial kernel scaffold; baseline (speedup 1.0000x reference)
import jax
import jax.numpy as jnp
from jax.experimental import pallas as pl


def kernel(x, edge_index, W1, b1, W2, b2):
    raise NotImplementedError("write your pallas kernel here")


# speedup vs baseline: 21.8116x; 21.8116x over previous
import jax
import jax.numpy as jnp
from jax import lax
from jax.experimental import pallas as pl
from jax.experimental.pallas import tpu as pltpu
from jax.experimental.pallas import tpu_sc as plsc

_N = 10000
_E = 320000
_D = 128
_NC = 2
_NS = 16
_NW = _NC * _NS
_EPW = _E // _NW
_CHUNK = 80
_NCHUNK = _EPW // _CHUNK
_WB = 632
_WBL = _N - (_NS - 1) * _WB

_mesh = plsc.VectorSubcoreMesh(core_axis_name="c", subcore_axis_name="s")


def _deg_body(dst_hbm, ones_hbm, init_hbm, out_hbm, dst_v, ones_v, acc):
    cid = lax.axis_index("c")
    sid = lax.axis_index("s")
    wid = cid * _NS + sid

    @pl.when(sid == 0)
    def _():
        pltpu.sync_copy(init_hbm.at[cid], acc)

    pltpu.sync_copy(dst_hbm.at[wid], dst_v)
    pltpu.sync_copy(ones_hbm, ones_v)
    plsc.subcore_barrier()

    def chunk(j, carry):
        pltpu.sync_copy(ones_v, acc.at[dst_v.at[j]], add=True)
        return carry

    lax.fori_loop(0, _NCHUNK, chunk, 0)
    plsc.subcore_barrier()

    @pl.when(sid == 0)
    def _():
        pltpu.sync_copy(acc, out_hbm.at[cid, 0])


_deg_kernel = pl.kernel(
    _deg_body,
    out_type=jax.ShapeDtypeStruct((_NC, 1, _N), jnp.float32),
    mesh=_mesh,
    scratch_types=[
        pltpu.VMEM((_NCHUNK, _CHUNK), jnp.int32),
        pltpu.VMEM((_CHUNK,), jnp.float32),
        pltpu.VMEM_SHARED((_N,), jnp.float32),
    ],
)


def _agg_body(table_hbm, src_hbm, dst_hbm, zeros_hbm, out_hbm,
              src_v, dst_v, rows_v, sem, acc):
    cid = lax.axis_index("c")
    sid = lax.axis_index("s")
    wid = cid * _NS + sid

    @pl.when(jnp.logical_and(sid == 0, cid == 0))
    def _():
        pltpu.sync_copy(table_hbm, acc)

    @pl.when(jnp.logical_and(sid == 0, cid == 1))
    def _():
        pltpu.sync_copy(zeros_hbm, acc)

    pltpu.sync_copy(src_hbm.at[wid], src_v)
    pltpu.sync_copy(dst_hbm.at[wid], dst_v)
    plsc.subcore_barrier()

    def chunk(j, carry):
        pltpu.async_copy(table_hbm.at[src_v.at[j]], rows_v, sem).wait()
        pltpu.sync_copy(rows_v, acc.at[dst_v.at[j]], add=True)
        return carry

    lax.fori_loop(0, _NCHUNK, chunk, 0)
    plsc.subcore_barrier()

    @pl.when(sid < _NS - 1)
    def _():
        pltpu.sync_copy(acc.at[pl.ds(sid * _WB, _WB)],
                        out_hbm.at[cid, pl.ds(sid * _WB, _WB)])

    @pl.when(sid == _NS - 1)
    def _():
        pltpu.sync_copy(acc.at[pl.ds((_NS - 1) * _WB, _WBL)],
                        out_hbm.at[cid, pl.ds((_NS - 1) * _WB, _WBL)])


_agg_kernel = pl.kernel(
    _agg_body,
    out_type=jax.ShapeDtypeStruct((_NC, _N, _D), jnp.float32),
    mesh=_mesh,
    scratch_types=[
        pltpu.VMEM((_NCHUNK, _CHUNK), jnp.int32),
        pltpu.VMEM((_NCHUNK, _CHUNK), jnp.int32),
        pltpu.VMEM((_CHUNK, _D), jnp.float32),
        pltpu.SemaphoreType.DMA,
        pltpu.VMEM_SHARED((_N, _D), jnp.float32),
    ],
)


_BLK = 1000


def _prep_body(d0_ref, d1_ref, x_ref, xs_ref, dinv_ref):
    deg = d0_ref[...] + d1_ref[...]
    dinv = lax.rsqrt(deg)
    dinv_ref[...] = dinv
    xs_ref[...] = x_ref[...] * dinv


def _mm_body(p0_ref, p1_ref, dinv_ref, w1_ref, b1_ref, w2_ref, out_ref):
    t = p0_ref[...] + p1_ref[...]
    dinv = dinv_ref[...]
    a = jnp.dot(t, w1_ref[...], preferred_element_type=jnp.float32)
    h = jnp.maximum(a * dinv + b1_ref[...], 0.0)
    g = jnp.dot(h, w2_ref[...], preferred_element_type=jnp.float32)
    out_ref[...] = g * dinv


def _fin_body(q0_ref, q1_ref, dinv_ref, b2_ref, out_ref):
    out_ref[...] = (q0_ref[...] + q1_ref[...]) * dinv_ref[...] + b2_ref[...]


def _row_spec(cols):
    return pl.BlockSpec((_BLK, cols), lambda i: (i, 0))


def _full_spec(r, c):
    return pl.BlockSpec((r, c), lambda i: (0, 0))


_prep_call = pl.pallas_call(
    _prep_body,
    grid=(_N // _BLK,),
    in_specs=[_row_spec(1), _row_spec(1), _row_spec(_D)],
    out_specs=[_row_spec(_D), _row_spec(1)],
    out_shape=[
        jax.ShapeDtypeStruct((_N, _D), jnp.float32),
        jax.ShapeDtypeStruct((_N, 1), jnp.float32),
    ],
)

_mm_call = pl.pallas_call(
    _mm_body,
    grid=(_N // _BLK,),
    in_specs=[
        _row_spec(_D), _row_spec(_D), _row_spec(1),
        _full_spec(128, 256), _full_spec(1, 256), _full_spec(256, 128),
    ],
    out_specs=_row_spec(_D),
    out_shape=jax.ShapeDtypeStruct((_N, _D), jnp.float32),
)

_fin_call = pl.pallas_call(
    _fin_body,
    grid=(_N // _BLK,),
    in_specs=[_row_spec(_D), _row_spec(_D), _row_spec(1), _full_spec(1, _D)],
    out_specs=_row_spec(_D),
    out_shape=jax.ShapeDtypeStruct((_N, _D), jnp.float32),
)


def kernel(x, edge_index, W1, b1, W2, b2):
    ei = edge_index.astype(jnp.int32)
    src = ei[0].reshape(_NW, _NCHUNK, _CHUNK)
    dst = ei[1].reshape(_NW, _NCHUNK, _CHUNK)
    zeros_nd = jnp.zeros((_N, _D), jnp.float32)
    deg_init = jnp.stack(
        [jnp.ones((_N,), jnp.float32), jnp.zeros((_N,), jnp.float32)])
    ones_c = jnp.ones((_CHUNK,), jnp.float32)

    degp = _deg_kernel(dst, ones_c, deg_init)
    d0 = degp[0].reshape(_N, 1)
    d1 = degp[1].reshape(_N, 1)
    xs, dinv = _prep_call(d0, d1, x)

    p = _agg_kernel(xs, src, dst, zeros_nd)
    g2 = _mm_call(p[0], p[1], dinv, W1, b1.reshape(1, -1), W2)

    q = _agg_kernel(g2, src, dst, zeros_nd)
    out = _fin_call(q[0], q[1], dinv, b2.reshape(1, -1))
    return out
